# initial kernel scaffold (unmeasured)
import jax
import jax.numpy as jnp
from jax import lax
from jax.experimental import pallas as pl
from jax.experimental.pallas import tpu as pltpu

N_DEV = 8
M = 4096
K = 4096
N = 8192
BLK = M // N_DEV


def kernel(x, w_mat):
    def body(x_ref, w_ref, out_ref, recv_buf, wbuf, send_sems, recv_sems,
             w_sems):
        me = lax.axis_index("i")

        bsem = pltpu.get_barrier_semaphore()
        for i in range(1, N_DEV):
            peer = lax.rem(me + i, N_DEV)
            pl.semaphore_signal(
                bsem, inc=1, device_id=(peer,),
                device_id_type=pl.DeviceIdType.MESH,
            )
        pl.semaphore_wait(bsem, N_DEV - 1)

        sends = []
        for i in range(1, N_DEV):
            d = lax.rem(me + i, N_DEV)
            rdma = pltpu.make_async_remote_copy(
                src_ref=x_ref.at[pl.ds(d * BLK, BLK), :],
                dst_ref=recv_buf.at[N_DEV - i],
                send_sem=send_sems.at[i - 1],
                recv_sem=recv_sems.at[N_DEV - i],
                device_id=(d,),
                device_id_type=pl.DeviceIdType.MESH,
            )
            rdma.start()
            sends.append(rdma)

        def w_copy(t):
            j = lax.rem(me + t, N_DEV)
            return pltpu.make_async_copy(
                w_ref.at[pl.ds(j * BLK, BLK), :],
                wbuf.at[t % 2],
                w_sems.at[t % 2],
            )

        w_copy(0).start()

        for t in range(N_DEV):
            w_copy(t).wait()
            if t + 1 < N_DEV:
                w_copy(t + 1).start()
            if t == 0:
                xblk = x_ref[pl.ds(me * BLK, BLK), :]
            else:
                recv = pltpu.make_async_remote_copy(
                    src_ref=recv_buf.at[t],
                    dst_ref=recv_buf.at[t],
                    send_sem=send_sems.at[0],
                    recv_sem=recv_sems.at[t],
                    device_id=(me,),
                    device_id_type=pl.DeviceIdType.MESH,
                )
                recv.wait_recv()
                xblk = recv_buf[t]
            part = jnp.dot(xblk, wbuf[t % 2],
                           preferred_element_type=jnp.float32)
            if t == 0:
                out_ref[...] = part
            elif t == N_DEV - 1:
                out_ref[...] = jnp.maximum(out_ref[...] + part, 0.0)
            else:
                out_ref[...] = out_ref[...] + part

        for rdma in sends:
            rdma.wait_send()

    return pl.pallas_call(
        body,
        out_shape=jax.ShapeDtypeStruct((BLK, N), jnp.float32),
        in_specs=[
            pl.BlockSpec(memory_space=pltpu.VMEM),
            pl.BlockSpec(memory_space=pltpu.ANY),
        ],
        out_specs=pl.BlockSpec(memory_space=pltpu.VMEM),
        scratch_shapes=[
            pltpu.VMEM((N_DEV, BLK, BLK), jnp.float32),
            pltpu.VMEM((2, BLK, N), jnp.float32),
            pltpu.SemaphoreType.DMA((N_DEV - 1,)),
            pltpu.SemaphoreType.DMA((N_DEV,)),
            pltpu.SemaphoreType.DMA((2,)),
        ],
        compiler_params=pltpu.CompilerParams(collective_id=0),
    )(x, w_mat)


# baseline (device time: 136136 ns/iter reference)
import jax
import jax.numpy as jnp
from jax import lax
from jax.experimental import pallas as pl
from jax.experimental.pallas import tpu as pltpu

N_DEV = 8
M = 4096
K = 4096
N = 8192
N_HALF = N // 2
BLK = M // N_DEV


def kernel(x, w_mat):
    def body(x_ref, w_ref, out_ref, recv_buf, wbuf, send_sems, recv_sems,
             w_sems):
        me = lax.axis_index("i")

        bsem = pltpu.get_barrier_semaphore()
        for i in range(1, N_DEV):
            peer = lax.rem(me + i, N_DEV)
            pl.semaphore_signal(
                bsem, inc=1, device_id=(peer,),
                device_id_type=pl.DeviceIdType.MESH,
            )
        pl.semaphore_wait(bsem, N_DEV - 1)

        sends = []
        for i in range(1, N_DEV):
            d = lax.rem(me + i, N_DEV)
            rdma = pltpu.make_async_remote_copy(
                src_ref=x_ref.at[pl.ds(d * BLK, BLK), :],
                dst_ref=recv_buf.at[N_DEV - i],
                send_sem=send_sems.at[i - 1],
                recv_sem=recv_sems.at[N_DEV - i],
                device_id=(d,),
                device_id_type=pl.DeviceIdType.MESH,
            )
            rdma.start()
            sends.append(rdma)

        def w_copy(s):
            j = lax.rem(me + (s // 2), N_DEV)
            h = s % 2
            return pltpu.make_async_copy(
                w_ref.at[pl.ds(j * BLK, BLK), pl.ds(h * N_HALF, N_HALF)],
                wbuf.at[s % 3],
                w_sems.at[s % 3],
            )

        w_copy(0).start()
        w_copy(1).start()

        for t in range(N_DEV):
            if t == 0:
                xblk = x_ref[pl.ds(me * BLK, BLK), :]
            else:
                recv = pltpu.make_async_remote_copy(
                    src_ref=recv_buf.at[t],
                    dst_ref=recv_buf.at[t],
                    send_sem=send_sems.at[0],
                    recv_sem=recv_sems.at[t],
                    device_id=(me,),
                    device_id_type=pl.DeviceIdType.MESH,
                )
                recv.wait_recv()
                xblk = recv_buf[t]
            for h in range(2):
                s = 2 * t + h
                w_copy(s).wait()
                part = jnp.dot(xblk, wbuf[s % 3],
                               preferred_element_type=jnp.float32)
                nsl = slice(h * N_HALF, (h + 1) * N_HALF)
                if t == 0:
                    out_ref[:, nsl] = part
                elif t == N_DEV - 1:
                    out_ref[:, nsl] = jnp.maximum(out_ref[:, nsl] + part, 0.0)
                else:
                    out_ref[:, nsl] = out_ref[:, nsl] + part
                if s + 2 < 2 * N_DEV:
                    w_copy(s + 2).start()

        for rdma in sends:
            rdma.wait_send()

    return pl.pallas_call(
        body,
        out_shape=jax.ShapeDtypeStruct((BLK, N), jnp.float32),
        in_specs=[
            pl.BlockSpec(memory_space=pltpu.VMEM),
            pl.BlockSpec(memory_space=pl.ANY),
        ],
        out_specs=pl.BlockSpec(memory_space=pltpu.VMEM),
        scratch_shapes=[
            pltpu.VMEM((N_DEV, BLK, BLK), jnp.float32),
            pltpu.VMEM((3, BLK, N_HALF), jnp.float32),
            pltpu.SemaphoreType.DMA((N_DEV - 1,)),
            pltpu.SemaphoreType.DMA((N_DEV,)),
            pltpu.SemaphoreType.DMA((3,)),
        ],
        compiler_params=pltpu.CompilerParams(
            collective_id=0,
            vmem_limit_bytes=63 * 1024 * 1024,
        ),
    )(x, w_mat)


# device time: 78284 ns/iter; 1.7390x vs baseline; 1.7390x over previous
import jax
import jax.numpy as jnp
from jax import lax
from jax.experimental import pallas as pl
from jax.experimental.pallas import tpu as pltpu

N_DEV = 8
M = 4096
K = 4096
N = 8192
N_HALF = N // 2
BLK = M // N_DEV


def kernel(x, w_mat):
    def body(x_ref, w_ref, out_ref, xs_ref, recv_buf, wbuf, send_sems,
             recv_sems, w_sems):
        me = lax.axis_index("i")

        bsem = pltpu.get_barrier_semaphore()
        for i in range(1, N_DEV):
            peer = lax.rem(me + i, N_DEV)
            pl.semaphore_signal(
                bsem, inc=1, device_id=(peer,),
                device_id_type=pl.DeviceIdType.MESH,
            )
        pl.semaphore_wait(bsem, N_DEV - 1)

        xs_ref[...] = x_ref[...].astype(jnp.bfloat16)

        sends = []
        for i in range(1, N_DEV):
            d = lax.rem(me - i + N_DEV, N_DEV)
            rdma = pltpu.make_async_remote_copy(
                src_ref=xs_ref.at[pl.ds(d * BLK, BLK), :],
                dst_ref=recv_buf.at[i],
                send_sem=send_sems.at[i - 1],
                recv_sem=recv_sems.at[i],
                device_id=(d,),
                device_id_type=pl.DeviceIdType.MESH,
            )
            rdma.start()
            sends.append(rdma)

        def w_copy(s):
            j = lax.rem(me + (s // 2), N_DEV)
            h = s % 2
            return pltpu.make_async_copy(
                w_ref.at[pl.ds(j * BLK, BLK), pl.ds(h * N_HALF, N_HALF)],
                wbuf.at[s % 3],
                w_sems.at[s % 3],
            )

        w_copy(0).start()
        w_copy(1).start()

        for t in range(N_DEV):
            if t == 0:
                xblk = x_ref[pl.ds(me * BLK, BLK), :]
            else:
                recv = pltpu.make_async_remote_copy(
                    src_ref=recv_buf.at[t],
                    dst_ref=recv_buf.at[t],
                    send_sem=send_sems.at[0],
                    recv_sem=recv_sems.at[t],
                    device_id=(me,),
                    device_id_type=pl.DeviceIdType.MESH,
                )
                recv.wait_recv()
                xblk = recv_buf[t].astype(jnp.float32)
            for h in range(2):
                s = 2 * t + h
                w_copy(s).wait()
                part = jnp.dot(xblk, wbuf[s % 3],
                               preferred_element_type=jnp.float32)
                nsl = slice(h * N_HALF, (h + 1) * N_HALF)
                if t == 0:
                    out_ref[:, nsl] = part
                elif t == N_DEV - 1:
                    out_ref[:, nsl] = jnp.maximum(out_ref[:, nsl] + part, 0.0)
                else:
                    out_ref[:, nsl] = out_ref[:, nsl] + part
                if s + 2 < 2 * N_DEV:
                    w_copy(s + 2).start()

        for rdma in sends:
            rdma.wait_send()

    return pl.pallas_call(
        body,
        out_shape=jax.ShapeDtypeStruct((BLK, N), jnp.float32),
        in_specs=[
            pl.BlockSpec(memory_space=pltpu.VMEM),
            pl.BlockSpec(memory_space=pl.ANY),
        ],
        out_specs=pl.BlockSpec(memory_space=pltpu.VMEM),
        scratch_shapes=[
            pltpu.VMEM((K, BLK), jnp.bfloat16),
            pltpu.VMEM((N_DEV, BLK, BLK), jnp.bfloat16),
            pltpu.VMEM((3, BLK, N_HALF), jnp.float32),
            pltpu.SemaphoreType.DMA((N_DEV - 1,)),
            pltpu.SemaphoreType.DMA((N_DEV,)),
            pltpu.SemaphoreType.DMA((3,)),
        ],
        compiler_params=pltpu.CompilerParams(
            collective_id=0,
            vmem_limit_bytes=63 * 1024 * 1024,
        ),
    )(x, w_mat)


# device time: 76337 ns/iter; 1.7834x vs baseline; 1.0255x over previous
import jax
import jax.numpy as jnp
from jax import lax
from jax.experimental import pallas as pl
from jax.experimental.pallas import tpu as pltpu

N_DEV = 8
M = 4096
K = 4096
N = 8192
N_HALF = N // 2
BLK = M // N_DEV


def kernel(x, w_mat):
    def body(x_ref, w_ref, out_ref, xs_ref, recv_buf, wbuf, send_sems,
             recv_sems, w_sems):
        me = lax.axis_index("i")

        bsem = pltpu.get_barrier_semaphore()
        for i in range(1, N_DEV):
            peer = lax.rem(me + i, N_DEV)
            pl.semaphore_signal(
                bsem, inc=1, device_id=(peer,),
                device_id_type=pl.DeviceIdType.MESH,
            )
        pl.semaphore_wait(bsem, N_DEV - 1)

        xs_ref[...] = x_ref[...].astype(jnp.bfloat16)

        def start_send(i):
            d = lax.rem(me - i + N_DEV, N_DEV)
            rdma = pltpu.make_async_remote_copy(
                src_ref=xs_ref.at[pl.ds(d * BLK, BLK), :],
                dst_ref=recv_buf.at[i],
                send_sem=send_sems.at[i - 1],
                recv_sem=recv_sems.at[i],
                device_id=(d,),
                device_id_type=pl.DeviceIdType.MESH,
            )
            rdma.start()
            return rdma

        sends = [start_send(1), start_send(2)]

        def w_copy(s):
            j = lax.rem(me + (s // 2), N_DEV)
            h = s % 2
            return pltpu.make_async_copy(
                w_ref.at[pl.ds(j * BLK, BLK), pl.ds(h * N_HALF, N_HALF)],
                wbuf.at[s % 3],
                w_sems.at[s % 3],
            )

        w_copy(0).start()
        w_copy(1).start()

        for t in range(N_DEV):
            if t + 3 < N_DEV:
                sends.append(start_send(t + 3))
            if t == 0:
                xblk = x_ref[pl.ds(me * BLK, BLK), :]
            else:
                recv = pltpu.make_async_remote_copy(
                    src_ref=recv_buf.at[t],
                    dst_ref=recv_buf.at[t],
                    send_sem=send_sems.at[0],
                    recv_sem=recv_sems.at[t],
                    device_id=(me,),
                    device_id_type=pl.DeviceIdType.MESH,
                )
                recv.wait_recv()
                xblk = recv_buf[t].astype(jnp.float32)
            for h in range(2):
                s = 2 * t + h
                w_copy(s).wait()
                part = jnp.dot(xblk, wbuf[s % 3],
                               preferred_element_type=jnp.float32)
                nsl = slice(h * N_HALF, (h + 1) * N_HALF)
                if t == 0:
                    out_ref[:, nsl] = part
                elif t == N_DEV - 1:
                    out_ref[:, nsl] = jnp.maximum(out_ref[:, nsl] + part, 0.0)
                else:
                    out_ref[:, nsl] = out_ref[:, nsl] + part
                if s + 2 < 2 * N_DEV:
                    w_copy(s + 2).start()

        for rdma in sends:
            rdma.wait_send()

    return pl.pallas_call(
        body,
        out_shape=jax.ShapeDtypeStruct((BLK, N), jnp.float32),
        in_specs=[
            pl.BlockSpec(memory_space=pltpu.VMEM),
            pl.BlockSpec(memory_space=pl.ANY),
        ],
        out_specs=pl.BlockSpec(memory_space=pltpu.VMEM),
        scratch_shapes=[
            pltpu.VMEM((K, BLK), jnp.bfloat16),
            pltpu.VMEM((N_DEV, BLK, BLK), jnp.bfloat16),
            pltpu.VMEM((3, BLK, N_HALF), jnp.float32),
            pltpu.SemaphoreType.DMA((N_DEV - 1,)),
            pltpu.SemaphoreType.DMA((N_DEV,)),
            pltpu.SemaphoreType.DMA((3,)),
        ],
        compiler_params=pltpu.CompilerParams(
            collective_id=0,
            vmem_limit_bytes=63 * 1024 * 1024,
        ),
    )(x, w_mat)
